# baseline (speedup 1.0000x reference)
import functools

import jax
import jax.numpy as jnp
from jax import lax
from jax.experimental import pallas as pl
from jax.experimental.pallas import tpu as pltpu
from jax.experimental.pallas import tpu_sc as plsc

N = 10000
E = 320000
DN = 128
DE = 16

NC = 2
NS = 16
L = 16
NW = NC * NS
EPW = E // NW
C = 40
NCH = EPW // C
KG = 5
NSLAB = 5
SLAB = NCH // NSLAB
CS = 40
NCHS = EPW // CS
KS = 2
SLABS = NCHS // NSLAB

_mesh = lambda: plsc.VectorSubcoreMesh(core_axis_name="c", subcore_axis_name="s")


def _sc_gather_sum(pab, ii_slabs, jj_slabs):
    @functools.partial(
        pl.kernel,
        mesh=_mesh(),
        out_type=jax.ShapeDtypeStruct((E, 64), jnp.float32),
        scratch_types=(
            [pltpu.VMEM((SLAB, C), jnp.int32)] * 2
            + [pltpu.VMEM((C, DN), jnp.float32)] * (2 * KG)
            + [pltpu.VMEM((C, 64), jnp.float32)] * KG
            + [pltpu.SemaphoreType.DMA] * (3 * KG)
        ),
    )
    def k(pab_h, *args):
        ii_h = args[0:NSLAB]
        jj_h = args[NSLAB:2 * NSLAB]
        s_h = args[2 * NSLAB]
        iv, jv = args[2 * NSLAB + 1:2 * NSLAB + 3]
        bufs = args[2 * NSLAB + 3:]
        ab = bufs[0:KG]
        bb = bufs[KG:2 * KG]
        sb = bufs[2 * KG:3 * KG]
        sga = bufs[3 * KG:4 * KG]
        sgb = bufs[4 * KG:5 * KG]
        sst = bufs[5 * KG:6 * KG]
        wid = lax.axis_index("c") * NS + lax.axis_index("s")
        ebase = wid * EPW

        for sl in range(NSLAB):
            pltpu.sync_copy(ii_h[sl].at[wid], iv)
            pltpu.sync_copy(jj_h[sl].at[wid], jv)
            cbase = ebase + sl * SLAB * C

            def group(g, carry, cbase=cbase):
                c0 = g * KG
                hs = []
                for b in range(KG):
                    ha = pltpu.async_copy(pab_h.at[iv.at[c0 + b]], ab[b], sga[b])
                    hb = pltpu.async_copy(pab_h.at[jv.at[c0 + b]], bb[b], sgb[b])
                    hs.append((ha, hb))
                sh = []
                for b in range(KG):
                    ha, hb = hs[b]
                    ha.wait()
                    hb.wait()

                    def add_row(r, c2, b=b):
                        for g2 in range(64 // L):
                            sl2 = pl.ds(g2 * L, L)
                            sb[b][r, sl2] = (ab[b][r, sl2]
                                             + bb[b][r, pl.ds(64 + g2 * L, L)])
                        return c2

                    lax.fori_loop(0, C, add_row, 0)
                    sh.append(pltpu.async_copy(
                        sb[b], s_h.at[pl.ds(cbase + (c0 + b) * C, C)], sst[b]))
                for x in sh:
                    x.wait()
                return carry

            lax.fori_loop(0, SLAB // KG, group, 0)

    return k(pab, *ii_slabs, *jj_slabs)


def _sc_scatter_msg(q, t, idx_slabs):
    @functools.partial(
        pl.kernel,
        mesh=_mesh(),
        out_type=jax.ShapeDtypeStruct((2 * N, DN), jnp.float32),
        scratch_types=(
            [pltpu.VMEM((SLABS, CS), jnp.int32)]
            + [pltpu.VMEM((CS, DN), jnp.float32)] * (2 * KS)
            + [pltpu.VMEM((16, DN), jnp.float32)]
            + [pltpu.VMEM_SHARED((N, DN), jnp.float32)]
            + [pltpu.SemaphoreType.DMA] * (3 * KS)
        ),
    )
    def k(q_h, t_h, ii0, ii1, ii2, ii3_, ii4, out_h, iv, *bufs):
        ii_slabs = (ii0, ii1, ii2, ii3_, ii4)
        tb = bufs[0:KS]
        qb = bufs[KS:2 * KS]
        zbuf = bufs[2 * KS]
        acc = bufs[2 * KS + 1]
        st = bufs[2 * KS + 2:3 * KS + 2]
        sq = bufs[3 * KS + 2:4 * KS + 2]
        ss = bufs[4 * KS + 2:5 * KS + 2]
        cid = lax.axis_index("c")
        sid = lax.axis_index("s")
        wid = cid * NS + sid
        ebase = wid * EPW

        def zrow(r, c2):
            def zcol(qq, c3):
                zbuf[r, pl.ds(qq * L, L)] = jnp.zeros((L,), jnp.float32)
                return c3
            return lax.fori_loop(0, DN // L, zcol, c2)

        lax.fori_loop(0, 16, zrow, 0)

        @pl.when(sid < NS - 1)
        def _zero_main():
            def zblk(b, carry):
                pltpu.sync_copy(zbuf, acc.at[pl.ds(sid * 624 + b * 16, 16)])
                return carry
            lax.fori_loop(0, 39, zblk, 0)

        @pl.when(sid == NS - 1)
        def _zero_tail():
            def zblk(b, carry):
                pltpu.sync_copy(zbuf, acc.at[pl.ds(9360 + b * 16, 16)])
                return carry
            lax.fori_loop(0, 40, zblk, 0)

        plsc.subcore_barrier()

        for sl in range(NSLAB):
            pltpu.sync_copy(ii_slabs[sl].at[wid], iv)
            cbase = ebase + sl * SLABS * CS

            def group(g, carry, cbase=cbase):
                c0 = g * KS
                hs = []
                for b in range(KS):
                    ht = pltpu.async_copy(
                        t_h.at[pl.ds(cbase + (c0 + b) * CS, CS)], tb[b], st[b])
                    hq = pltpu.async_copy(q_h.at[iv.at[c0 + b]], qb[b], sq[b])
                    hs.append((ht, hq))
                sh = []
                for b in range(KS):
                    ht, hq = hs[b]
                    ht.wait()
                    hq.wait()

                    def mrow(r, c2, b=b):
                        for qq in range(DN // L):
                            sl2 = pl.ds(qq * L, L)
                            tb[b][r, sl2] = jnp.maximum(
                                tb[b][r, sl2] + qb[b][r, sl2], 0.0)
                        return c2

                    lax.fori_loop(0, CS, mrow, 0)
                    sh.append(pltpu.async_copy(
                        tb[b], acc.at[iv.at[c0 + b]], ss[b], add=True))
                for x in sh:
                    x.wait()
                return carry

            lax.fori_loop(0, SLABS // KS, group, 0)
        plsc.subcore_barrier()

        @pl.when(sid < NS - 1)
        def _exp_main():
            def eblk(b, carry):
                row0 = sid * 624 + b * 104
                pltpu.sync_copy(acc.at[pl.ds(row0, 104)],
                                out_h.at[pl.ds(cid * N + row0, 104)])
                return carry
            lax.fori_loop(0, 6, eblk, 0)

        @pl.when(sid == NS - 1)
        def _exp_tail():
            def eblk(b, carry):
                row0 = 9360 + b * 80
                pltpu.sync_copy(acc.at[pl.ds(row0, 80)],
                                out_h.at[pl.ds(cid * N + row0, 80)])
                return carry
            lax.fori_loop(0, 8, eblk, 0)

    return k(q, t, *idx_slabs)


_NBLK = 2000
_EBLK = 4000


def _tc_node(x, ne_w1, ne_b1, ne_w2, ne_b2, wab, wq):
    def body(x_r, w1_r, b1_r, w2_r, b2_r, wab_r, wq_r, pab_r, q_r):
        h = jnp.maximum(jnp.dot(x_r[...], w1_r[...],
                                preferred_element_type=jnp.float32) + b1_r[...], 0.0)
        nf = jnp.dot(h, w2_r[...], preferred_element_type=jnp.float32) + b2_r[...]
        pab_r[...] = jnp.dot(nf, wab_r[...], preferred_element_type=jnp.float32)
        q_r[...] = jnp.dot(nf, wq_r[...], preferred_element_type=jnp.float32)

    full = lambda s: pl.BlockSpec(s, lambda i: (0, 0))
    return pl.pallas_call(
        body,
        grid=(N // _NBLK,),
        in_specs=[
            pl.BlockSpec((_NBLK, DN), lambda i: (i, 0)),
            full((DN, 128)), full((1, 128)), full((128, DN)), full((1, DN)),
            full((DN, DN)), full((DN, DN)),
        ],
        out_specs=[
            pl.BlockSpec((_NBLK, DN), lambda i: (i, 0)),
            pl.BlockSpec((_NBLK, DN), lambda i: (i, 0)),
        ],
        out_shape=[
            jax.ShapeDtypeStruct((N, DN), jnp.float32),
            jax.ShapeDtypeStruct((N, DN), jnp.float32),
        ],
    )(x, ne_w1, ne_b1, ne_w2, ne_b2, wab, wq)


def _tc_edge_embed(edge_attr, ee_w1, ee_b1, ee_w2, ee_b2):
    def body(ea_r, w1_r, b1_r, w2_r, b2_r, ef0_r):
        e1 = jnp.maximum(jnp.dot(ea_r[...], w1_r[...],
                                 preferred_element_type=jnp.float32) + b1_r[...], 0.0)
        ef0_r[...] = jnp.dot(e1, w2_r[...],
                             preferred_element_type=jnp.float32) + b2_r[...]

    full = lambda s: pl.BlockSpec(s, lambda i: (0, 0))
    return pl.pallas_call(
        body,
        grid=(E // _EBLK,),
        in_specs=[
            pl.BlockSpec((_EBLK, DE), lambda i: (i, 0)),
            full((DE, 64)), full((1, 64)), full((64, DE)), full((1, DE)),
        ],
        out_specs=pl.BlockSpec((_EBLK, DE), lambda i: (i, 0)),
        out_shape=jax.ShapeDtypeStruct((E, DE), jnp.float32),
    )(edge_attr, ee_w1, ee_b1, ee_w2, ee_b2)


def _tc_edge_first(s0, ef0, wc, me_b1, me_w2, me_b2, wt, mn_b1):
    def body(s_r, ef0_r, wc_r, mb1_r, mw2_r, mb2_r, wt_r, nb1_r, ef_r, t_r):
        h1 = jnp.maximum(s_r[...].astype(jnp.float32)
                         + jnp.dot(ef0_r[...], wc_r[...],
                                   preferred_element_type=jnp.float32)
                         + mb1_r[...], 0.0)
        ef1 = jnp.maximum(jnp.dot(h1, mw2_r[...],
                                  preferred_element_type=jnp.float32) + mb2_r[...], 0.0)
        ef_r[...] = ef1
        t_r[...] = (jnp.dot(ef1, wt_r[...], preferred_element_type=jnp.float32)
                    + nb1_r[...])

    full = lambda s: pl.BlockSpec(s, lambda i: (0, 0))
    return pl.pallas_call(
        body,
        grid=(E // _EBLK,),
        in_specs=[
            pl.BlockSpec((_EBLK, 64), lambda i: (i, 0)),
            pl.BlockSpec((_EBLK, DE), lambda i: (i, 0)),
            full((DE, 64)), full((1, 64)), full((64, DE)), full((1, DE)),
            full((DE, DN)), full((1, DN)),
        ],
        out_specs=[
            pl.BlockSpec((_EBLK, DE), lambda i: (i, 0)),
            pl.BlockSpec((_EBLK, DN), lambda i: (i, 0)),
        ],
        out_shape=[
            jax.ShapeDtypeStruct((E, DE), jnp.float32),
            jax.ShapeDtypeStruct((E, DN), jnp.float32),
        ],
    )(s0, ef0, wc, me_b1, me_w2, me_b2, wt, mn_b1)


def _tc_combine(parts, wab):
    def body(p0_r, p1_r, wab_r, pab_r):
        nf = p0_r[...] + p1_r[...]
        pab_r[...] = jnp.dot(nf, wab_r[...], preferred_element_type=jnp.float32)

    return pl.pallas_call(
        body,
        grid=(N // _NBLK,),
        in_specs=[
            pl.BlockSpec((_NBLK, DN), lambda i: (i, 0)),
            pl.BlockSpec((_NBLK, DN), lambda i: (i + N // _NBLK, 0)),
            pl.BlockSpec((DN, DN), lambda i: (0, 0)),
        ],
        out_specs=pl.BlockSpec((_NBLK, DN), lambda i: (i, 0)),
        out_shape=jax.ShapeDtypeStruct((N, DN), jnp.float32),
    )(parts, parts, wab)


def _tc_edge_last(s1, ef1, wc, me_b1, me_w2, me_b2, cl_w1, cl_b1, cl_w2, cl_b2):
    def body(s_r, ef_r, wc_r, mb1_r, mw2_r, mb2_r, cw1_r, cb1_r, cw2_r, cb2_r,
             o_r):
        h1 = jnp.maximum(s_r[...].astype(jnp.float32)
                         + jnp.dot(ef_r[...], wc_r[...],
                                   preferred_element_type=jnp.float32)
                         + mb1_r[...], 0.0)
        ef2 = jnp.maximum(jnp.dot(h1, mw2_r[...],
                                  preferred_element_type=jnp.float32) + mb2_r[...], 0.0)
        c1 = jnp.maximum(jnp.dot(ef2, cw1_r[...],
                                 preferred_element_type=jnp.float32) + cb1_r[...], 0.0)
        o_r[...] = jnp.dot(c1, cw2_r[...],
                           preferred_element_type=jnp.float32) + cb2_r[...]

    full = lambda s: pl.BlockSpec(s, lambda i: (0, 0))
    return pl.pallas_call(
        body,
        grid=(E // _EBLK,),
        in_specs=[
            pl.BlockSpec((_EBLK, 64), lambda i: (i, 0)),
            pl.BlockSpec((_EBLK, DE), lambda i: (i, 0)),
            full((DE, 64)), full((1, 64)), full((64, DE)), full((1, DE)),
            full((DE, 32)), full((1, 32)), full((32, 1)), full((1, 1)),
        ],
        out_specs=pl.BlockSpec((_EBLK, 1), lambda i: (i, 0)),
        out_shape=jax.ShapeDtypeStruct((E, 1), jnp.float32),
    )(s1, ef1, wc, me_b1, me_w2, me_b2, cl_w1, cl_b1, cl_w2, cl_b2)


def kernel(x, edge_attr, edge_index, ne_w1, ne_b1, ne_w2, ne_b2,
           ee_w1, ee_b1, ee_w2, ee_b2, me_w1, me_b1, me_w2, me_b2,
           mn_w1, mn_b1, cl_w1, cl_b1, cl_w2, cl_b2):
    idx_j4 = edge_index[0].reshape(NW, NSLAB, SLAB, C)
    idx_i4 = edge_index[1].reshape(NW, NSLAB, SLAB, C)
    jj_slabs = tuple(idx_j4[:, s] for s in range(NSLAB))
    ii_slabs = tuple(idx_i4[:, s] for s in range(NSLAB))
    idx_i4s = edge_index[1].reshape(NW, NSLAB, SLABS, CS)
    ii_slabs_s = tuple(idx_i4s[:, s] for s in range(NSLAB))

    wab = jnp.concatenate([me_w1[:DN], me_w1[DN:2 * DN]], axis=1)
    wc = me_w1[2 * DN:]
    wq = mn_w1[:DN]
    wt = mn_w1[DN:]
    r2 = lambda b: b.reshape(1, -1)

    pab0, q0 = _tc_node(x, ne_w1, r2(ne_b1), ne_w2, r2(ne_b2), wab, wq)
    ef0 = _tc_edge_embed(edge_attr, ee_w1, r2(ee_b1), ee_w2, r2(ee_b2))
    s0 = _sc_gather_sum(pab0, ii_slabs, jj_slabs)
    ef1, t0 = _tc_edge_first(s0, ef0, wc, r2(me_b1), me_w2, r2(me_b2),
                             wt, r2(mn_b1))
    parts = _sc_scatter_msg(q0, t0, ii_slabs_s)
    pab1 = _tc_combine(parts, wab)
    s1 = _sc_gather_sum(pab1, ii_slabs, jj_slabs)
    out = _tc_edge_last(s1, ef1, wc, r2(me_b1), me_w2, r2(me_b2),
                        cl_w1, r2(cl_b1), cl_w2, r2(cl_b2))
    return out


# speedup vs baseline: 1.0828x; 1.0828x over previous
import functools

import jax
import jax.numpy as jnp
from jax import lax
from jax.experimental import pallas as pl
from jax.experimental.pallas import tpu as pltpu
from jax.experimental.pallas import tpu_sc as plsc

N = 10000
E = 320000
DN = 128
DE = 16

NC = 2
NS = 16
L = 16
NW = NC * NS
EPW = E // NW
C = 40
NCH = EPW // C
KG = 5
NSLAB = 5
SLAB = NCH // NSLAB
CS = 40
NCHS = EPW // CS
KS = 2
SLABS = NCHS // NSLAB

_mesh = lambda: plsc.VectorSubcoreMesh(core_axis_name="c", subcore_axis_name="s")


def _sc_gather_sum(pab, ii_slabs, jj_slabs):
    @functools.partial(
        pl.kernel,
        mesh=_mesh(),
        out_type=jax.ShapeDtypeStruct((E, 64), jnp.float32),
        scratch_types=(
            [pltpu.VMEM((SLAB, C), jnp.int32)] * 2
            + [pltpu.VMEM((C, DN), jnp.float32)] * (2 * KG)
            + [pltpu.VMEM((C, 64), jnp.float32)] * KG
            + [pltpu.SemaphoreType.DMA] * (3 * KG)
        ),
    )
    def k(pab_h, *args):
        ii_h = args[0:NSLAB]
        jj_h = args[NSLAB:2 * NSLAB]
        s_h = args[2 * NSLAB]
        iv, jv = args[2 * NSLAB + 1:2 * NSLAB + 3]
        bufs = args[2 * NSLAB + 3:]
        ab = bufs[0:KG]
        bb = bufs[KG:2 * KG]
        sb = bufs[2 * KG:3 * KG]
        sga = bufs[3 * KG:4 * KG]
        sgb = bufs[4 * KG:5 * KG]
        sst = bufs[5 * KG:6 * KG]
        wid = lax.axis_index("c") * NS + lax.axis_index("s")
        ebase = wid * EPW

        for sl in range(NSLAB):
            pltpu.sync_copy(ii_h[sl].at[wid], iv)
            pltpu.sync_copy(jj_h[sl].at[wid], jv)
            cbase = ebase + sl * SLAB * C

            def group(g, carry, cbase=cbase):
                c0 = g * KG
                hs = []
                for b in range(KG):
                    ha = pltpu.async_copy(pab_h.at[iv.at[c0 + b]], ab[b], sga[b])
                    hb = pltpu.async_copy(pab_h.at[jv.at[c0 + b]], bb[b], sgb[b])
                    hs.append((ha, hb))
                sh = []
                for b in range(KG):
                    ha, hb = hs[b]
                    ha.wait()
                    hb.wait()

                    def add_row(r, c2, b=b):
                        for g2 in range(64 // L):
                            sl2 = pl.ds(g2 * L, L)
                            sb[b][r, sl2] = (ab[b][r, sl2]
                                             + bb[b][r, pl.ds(64 + g2 * L, L)])
                        return c2

                    lax.fori_loop(0, C, add_row, 0)
                    sh.append(pltpu.async_copy(
                        sb[b], s_h.at[pl.ds(cbase + (c0 + b) * C, C)], sst[b]))
                for x in sh:
                    x.wait()
                return carry

            lax.fori_loop(0, SLAB // KG, group, 0)

    return k(pab, *ii_slabs, *jj_slabs)


def _sc_scatter_msg(q, t, idx_slabs):
    @functools.partial(
        pl.kernel,
        mesh=_mesh(),
        out_type=jax.ShapeDtypeStruct((2 * N, DN), jnp.float32),
        scratch_types=(
            [pltpu.VMEM((SLABS, CS), jnp.int32)]
            + [pltpu.VMEM((CS, DN), jnp.float32)] * (2 * KS)
            + [pltpu.VMEM((16, DN), jnp.float32)]
            + [pltpu.VMEM_SHARED((N, DN), jnp.float32)]
            + [pltpu.SemaphoreType.DMA] * (3 * KS)
        ),
    )
    def k(q_h, t_h, ii0, ii1, ii2, ii3_, ii4, out_h, iv, *bufs):
        ii_slabs = (ii0, ii1, ii2, ii3_, ii4)
        tb = bufs[0:KS]
        qb = bufs[KS:2 * KS]
        zbuf = bufs[2 * KS]
        acc = bufs[2 * KS + 1]
        st = bufs[2 * KS + 2:3 * KS + 2]
        sq = bufs[3 * KS + 2:4 * KS + 2]
        ss = bufs[4 * KS + 2:5 * KS + 2]
        cid = lax.axis_index("c")
        sid = lax.axis_index("s")
        wid = cid * NS + sid
        ebase = wid * EPW

        def zrow(r, c2):
            def zcol(qq, c3):
                zbuf[r, pl.ds(qq * L, L)] = jnp.zeros((L,), jnp.float32)
                return c3
            return lax.fori_loop(0, DN // L, zcol, c2)

        lax.fori_loop(0, 16, zrow, 0)

        @pl.when(sid < NS - 1)
        def _zero_main():
            def zblk(b, carry):
                pltpu.sync_copy(zbuf, acc.at[pl.ds(sid * 624 + b * 16, 16)])
                return carry
            lax.fori_loop(0, 39, zblk, 0)

        @pl.when(sid == NS - 1)
        def _zero_tail():
            def zblk(b, carry):
                pltpu.sync_copy(zbuf, acc.at[pl.ds(9360 + b * 16, 16)])
                return carry
            lax.fori_loop(0, 40, zblk, 0)

        plsc.subcore_barrier()

        for sl in range(NSLAB):
            pltpu.sync_copy(ii_slabs[sl].at[wid], iv)
            cbase = ebase + sl * SLABS * CS

            def group(g, carry, cbase=cbase):
                c0 = g * KS
                hs = []
                for b in range(KS):
                    ht = pltpu.async_copy(
                        t_h.at[pl.ds(cbase + (c0 + b) * CS, CS)], tb[b], st[b])
                    hq = pltpu.async_copy(q_h.at[iv.at[c0 + b]], qb[b], sq[b])
                    hs.append((ht, hq))
                sh = []
                for b in range(KS):
                    ht, hq = hs[b]
                    ht.wait()
                    hq.wait()

                    def mrow(r, c2, b=b):
                        for qq in range(DN // L):
                            sl2 = pl.ds(qq * L, L)
                            tb[b][r, sl2] = jnp.maximum(
                                tb[b][r, sl2] + qb[b][r, sl2], 0.0)
                        return c2

                    lax.fori_loop(0, CS, mrow, 0)
                    sh.append(pltpu.async_copy(
                        tb[b], acc.at[iv.at[c0 + b]], ss[b], add=True))
                for x in sh:
                    x.wait()
                return carry

            lax.fori_loop(0, SLABS // KS, group, 0)
        plsc.subcore_barrier()

        @pl.when(sid < NS - 1)
        def _exp_main():
            def eblk(b, carry):
                row0 = sid * 624 + b * 104
                pltpu.sync_copy(acc.at[pl.ds(row0, 104)],
                                out_h.at[pl.ds(cid * N + row0, 104)])
                return carry
            lax.fori_loop(0, 6, eblk, 0)

        @pl.when(sid == NS - 1)
        def _exp_tail():
            def eblk(b, carry):
                row0 = 9360 + b * 80
                pltpu.sync_copy(acc.at[pl.ds(row0, 80)],
                                out_h.at[pl.ds(cid * N + row0, 80)])
                return carry
            lax.fori_loop(0, 8, eblk, 0)

    return k(q, t, *idx_slabs)


_NBLK = 2000
_EBLK = 8000


def _tc_node(x, ne_w1, ne_b1, ne_w2, ne_b2, wab, wq):
    def body(x_r, w1_r, b1_r, w2_r, b2_r, wab_r, wq_r, pab_r, q_r):
        h = jnp.maximum(jnp.dot(x_r[...], w1_r[...],
                                preferred_element_type=jnp.float32) + b1_r[...], 0.0)
        nf = jnp.dot(h, w2_r[...], preferred_element_type=jnp.float32) + b2_r[...]
        pab_r[...] = jnp.dot(nf, wab_r[...], preferred_element_type=jnp.float32)
        q_r[...] = jnp.dot(nf, wq_r[...], preferred_element_type=jnp.float32)

    full = lambda s: pl.BlockSpec(s, lambda i: (0, 0))
    return pl.pallas_call(
        body,
        grid=(N // _NBLK,),
        in_specs=[
            pl.BlockSpec((_NBLK, DN), lambda i: (i, 0)),
            full((DN, 128)), full((1, 128)), full((128, DN)), full((1, DN)),
            full((DN, DN)), full((DN, DN)),
        ],
        out_specs=[
            pl.BlockSpec((_NBLK, DN), lambda i: (i, 0)),
            pl.BlockSpec((_NBLK, DN), lambda i: (i, 0)),
        ],
        out_shape=[
            jax.ShapeDtypeStruct((N, DN), jnp.float32),
            jax.ShapeDtypeStruct((N, DN), jnp.float32),
        ],
    )(x, ne_w1, ne_b1, ne_w2, ne_b2, wab, wq)


def _tc_edge_first(s0, edge_attr, ee_w1, ee_b1, ee_w2, ee_b2,
                   wc, me_b1, me_w2, me_b2, wt, mn_b1):
    def body(s_r, ea_r, w1_r, b1_r, w2_r, b2_r, wc_r, mb1_r, mw2_r, mb2_r,
             wt_r, nb1_r, ef_r, t_r):
        e1 = jnp.maximum(jnp.dot(ea_r[...], w1_r[...],
                                 preferred_element_type=jnp.float32) + b1_r[...], 0.0)
        ef0 = jnp.dot(e1, w2_r[...], preferred_element_type=jnp.float32) + b2_r[...]
        h1 = jnp.maximum(s_r[...].astype(jnp.float32)
                         + jnp.dot(ef0, wc_r[...],
                                   preferred_element_type=jnp.float32)
                         + mb1_r[...], 0.0)
        ef1 = jnp.maximum(jnp.dot(h1, mw2_r[...],
                                  preferred_element_type=jnp.float32) + mb2_r[...], 0.0)
        ef_r[...] = ef1
        t_r[...] = (jnp.dot(ef1, wt_r[...], preferred_element_type=jnp.float32)
                    + nb1_r[...])

    full = lambda s: pl.BlockSpec(s, lambda i: (0, 0))
    return pl.pallas_call(
        body,
        grid=(E // _EBLK,),
        in_specs=[
            pl.BlockSpec((_EBLK, 64), lambda i: (i, 0)),
            pl.BlockSpec((_EBLK, DE), lambda i: (i, 0)),
            full((DE, 64)), full((1, 64)), full((64, DE)), full((1, DE)),
            full((DE, 64)), full((1, 64)), full((64, DE)), full((1, DE)),
            full((DE, DN)), full((1, DN)),
        ],
        out_specs=[
            pl.BlockSpec((_EBLK, DE), lambda i: (i, 0)),
            pl.BlockSpec((_EBLK, DN), lambda i: (i, 0)),
        ],
        out_shape=[
            jax.ShapeDtypeStruct((E, DE), jnp.float32),
            jax.ShapeDtypeStruct((E, DN), jnp.float32),
        ],
    )(s0, edge_attr, ee_w1, ee_b1, ee_w2, ee_b2, wc, me_b1, me_w2, me_b2,
      wt, mn_b1)


def _tc_combine(parts, wab):
    def body(p0_r, p1_r, wab_r, pab_r):
        nf = p0_r[...] + p1_r[...]
        pab_r[...] = jnp.dot(nf, wab_r[...], preferred_element_type=jnp.float32)

    return pl.pallas_call(
        body,
        grid=(N // _NBLK,),
        in_specs=[
            pl.BlockSpec((_NBLK, DN), lambda i: (i, 0)),
            pl.BlockSpec((_NBLK, DN), lambda i: (i + N // _NBLK, 0)),
            pl.BlockSpec((DN, DN), lambda i: (0, 0)),
        ],
        out_specs=pl.BlockSpec((_NBLK, DN), lambda i: (i, 0)),
        out_shape=jax.ShapeDtypeStruct((N, DN), jnp.float32),
    )(parts, parts, wab)


def _tc_edge_last(s1, ef1, wc, me_b1, me_w2, me_b2, cl_w1, cl_b1, cl_w2, cl_b2):
    def body(s_r, ef_r, wc_r, mb1_r, mw2_r, mb2_r, cw1_r, cb1_r, cw2_r, cb2_r,
             o_r):
        h1 = jnp.maximum(s_r[...].astype(jnp.float32)
                         + jnp.dot(ef_r[...], wc_r[...],
                                   preferred_element_type=jnp.float32)
                         + mb1_r[...], 0.0)
        ef2 = jnp.maximum(jnp.dot(h1, mw2_r[...],
                                  preferred_element_type=jnp.float32) + mb2_r[...], 0.0)
        c1 = jnp.maximum(jnp.dot(ef2, cw1_r[...],
                                 preferred_element_type=jnp.float32) + cb1_r[...], 0.0)
        o_r[...] = jnp.dot(c1, cw2_r[...],
                           preferred_element_type=jnp.float32) + cb2_r[...]

    full = lambda s: pl.BlockSpec(s, lambda i: (0, 0))
    return pl.pallas_call(
        body,
        grid=(E // _EBLK,),
        in_specs=[
            pl.BlockSpec((_EBLK, 64), lambda i: (i, 0)),
            pl.BlockSpec((_EBLK, DE), lambda i: (i, 0)),
            full((DE, 64)), full((1, 64)), full((64, DE)), full((1, DE)),
            full((DE, 32)), full((1, 32)), full((32, 1)), full((1, 1)),
        ],
        out_specs=pl.BlockSpec((_EBLK, 1), lambda i: (i, 0)),
        out_shape=jax.ShapeDtypeStruct((E, 1), jnp.float32),
    )(s1, ef1, wc, me_b1, me_w2, me_b2, cl_w1, cl_b1, cl_w2, cl_b2)


def kernel(x, edge_attr, edge_index, ne_w1, ne_b1, ne_w2, ne_b2,
           ee_w1, ee_b1, ee_w2, ee_b2, me_w1, me_b1, me_w2, me_b2,
           mn_w1, mn_b1, cl_w1, cl_b1, cl_w2, cl_b2):
    idx_j4 = edge_index[0].reshape(NW, NSLAB, SLAB, C)
    idx_i4 = edge_index[1].reshape(NW, NSLAB, SLAB, C)
    jj_slabs = tuple(idx_j4[:, s] for s in range(NSLAB))
    ii_slabs = tuple(idx_i4[:, s] for s in range(NSLAB))
    idx_i4s = edge_index[1].reshape(NW, NSLAB, SLABS, CS)
    ii_slabs_s = tuple(idx_i4s[:, s] for s in range(NSLAB))

    wab = jnp.concatenate([me_w1[:DN], me_w1[DN:2 * DN]], axis=1)
    wc = me_w1[2 * DN:]
    wq = mn_w1[:DN]
    wt = mn_w1[DN:]
    r2 = lambda b: b.reshape(1, -1)

    pab0, q0 = _tc_node(x, ne_w1, r2(ne_b1), ne_w2, r2(ne_b2), wab, wq)
    s0 = _sc_gather_sum(pab0, ii_slabs, jj_slabs)
    ef1, t0 = _tc_edge_first(s0, edge_attr, ee_w1, r2(ee_b1), ee_w2, r2(ee_b2),
                             wc, r2(me_b1), me_w2, r2(me_b2), wt, r2(mn_b1))
    parts = _sc_scatter_msg(q0, t0, ii_slabs_s)
    pab1 = _tc_combine(parts, wab)
    s1 = _sc_gather_sum(pab1, ii_slabs, jj_slabs)
    out = _tc_edge_last(s1, ef1, wc, r2(me_b1), me_w2, r2(me_b2),
                        cl_w1, r2(cl_b1), cl_w2, r2(cl_b2))
    return out


# speedup vs baseline: 1.0845x; 1.0016x over previous
import functools

import jax
import jax.numpy as jnp
from jax import lax
from jax.experimental import pallas as pl
from jax.experimental.pallas import tpu as pltpu
from jax.experimental.pallas import tpu_sc as plsc

N = 10000
E = 320000
DN = 128
DE = 16

NC = 2
NS = 16
L = 16
NW = NC * NS
EPW = E // NW
C = 40
NCH = EPW // C
KG = 5
NSLAB = 5
SLAB = NCH // NSLAB
CS = 40
NCHS = EPW // CS
KS = 2
SLABS = NCHS // NSLAB

_mesh = lambda: plsc.VectorSubcoreMesh(core_axis_name="c", subcore_axis_name="s")


def _sc_gather_sum(pab, ii_slabs, jj_slabs):
    @functools.partial(
        pl.kernel,
        mesh=_mesh(),
        out_type=jax.ShapeDtypeStruct((E, 64), jnp.float32),
        scratch_types=(
            [pltpu.VMEM((SLAB, C), jnp.int32)] * 2
            + [pltpu.VMEM((C, DN), jnp.float32)] * (2 * KG)
            + [pltpu.VMEM((C, 64), jnp.float32)] * KG
            + [pltpu.SemaphoreType.DMA] * (3 * KG)
        ),
    )
    def k(pab_h, *args):
        ii_h = args[0:NSLAB]
        jj_h = args[NSLAB:2 * NSLAB]
        s_h = args[2 * NSLAB]
        iv, jv = args[2 * NSLAB + 1:2 * NSLAB + 3]
        bufs = args[2 * NSLAB + 3:]
        ab = bufs[0:KG]
        bb = bufs[KG:2 * KG]
        sb = bufs[2 * KG:3 * KG]
        sga = bufs[3 * KG:4 * KG]
        sgb = bufs[4 * KG:5 * KG]
        sst = bufs[5 * KG:6 * KG]
        wid = lax.axis_index("c") * NS + lax.axis_index("s")
        ebase = wid * EPW

        for sl in range(NSLAB):
            pltpu.sync_copy(ii_h[sl].at[wid], iv)
            pltpu.sync_copy(jj_h[sl].at[wid], jv)
            cbase = ebase + sl * SLAB * C

            def group(g, carry, cbase=cbase):
                c0 = g * KG
                hs = []
                for b in range(KG):
                    ha = pltpu.async_copy(pab_h.at[iv.at[c0 + b]], ab[b], sga[b])
                    hb = pltpu.async_copy(pab_h.at[jv.at[c0 + b]], bb[b], sgb[b])
                    hs.append((ha, hb))
                sh = []
                for b in range(KG):
                    ha, hb = hs[b]
                    ha.wait()
                    hb.wait()

                    def add_row(r, c2, b=b):
                        for g2 in range(64 // L):
                            sl2 = pl.ds(g2 * L, L)
                            sb[b][r, sl2] = (ab[b][r, sl2]
                                             + bb[b][r, pl.ds(64 + g2 * L, L)])
                        return c2

                    lax.fori_loop(0, C, add_row, 0)
                    sh.append(pltpu.async_copy(
                        sb[b], s_h.at[pl.ds(cbase + (c0 + b) * C, C)], sst[b]))
                for x in sh:
                    x.wait()
                return carry

            lax.fori_loop(0, SLAB // KG, group, 0)

    return k(pab, *ii_slabs, *jj_slabs)


def _sc_scatter_msg(q, t, idx_slabs):
    @functools.partial(
        pl.kernel,
        mesh=_mesh(),
        out_type=jax.ShapeDtypeStruct((2 * N, DN), jnp.float32),
        scratch_types=(
            [pltpu.VMEM((SLABS, CS), jnp.int32)]
            + [pltpu.VMEM((CS, DN), jnp.float32)] * (2 * KS)
            + [pltpu.VMEM((16, DN), jnp.float32)]
            + [pltpu.VMEM_SHARED((N, DN), jnp.float32)]
            + [pltpu.SemaphoreType.DMA] * (3 * KS)
        ),
    )
    def k(q_h, t_h, ii0, ii1, ii2, ii3_, ii4, out_h, iv, *bufs):
        ii_slabs = (ii0, ii1, ii2, ii3_, ii4)
        tb = bufs[0:KS]
        qb = bufs[KS:2 * KS]
        zbuf = bufs[2 * KS]
        acc = bufs[2 * KS + 1]
        st = bufs[2 * KS + 2:3 * KS + 2]
        sq = bufs[3 * KS + 2:4 * KS + 2]
        ss = bufs[4 * KS + 2:5 * KS + 2]
        cid = lax.axis_index("c")
        sid = lax.axis_index("s")
        wid = cid * NS + sid
        ebase = wid * EPW

        def zrow(r, c2):
            def zcol(qq, c3):
                zbuf[r, pl.ds(qq * L, L)] = jnp.zeros((L,), jnp.float32)
                return c3
            return lax.fori_loop(0, DN // L, zcol, c2)

        lax.fori_loop(0, 16, zrow, 0)

        @pl.when(sid < NS - 1)
        def _zero_main():
            def zblk(b, carry):
                pltpu.sync_copy(zbuf, acc.at[pl.ds(sid * 624 + b * 16, 16)])
                return carry
            lax.fori_loop(0, 39, zblk, 0)

        @pl.when(sid == NS - 1)
        def _zero_tail():
            def zblk(b, carry):
                pltpu.sync_copy(zbuf, acc.at[pl.ds(9360 + b * 16, 16)])
                return carry
            lax.fori_loop(0, 40, zblk, 0)

        plsc.subcore_barrier()

        for sl in range(NSLAB):
            pltpu.sync_copy(ii_slabs[sl].at[wid], iv)
            cbase = ebase + sl * SLABS * CS

            def group(g, carry, cbase=cbase):
                c0 = g * KS
                hs = []
                for b in range(KS):
                    ht = pltpu.async_copy(
                        t_h.at[pl.ds(cbase + (c0 + b) * CS, CS)], tb[b], st[b])
                    hq = pltpu.async_copy(q_h.at[iv.at[c0 + b]], qb[b], sq[b])
                    hs.append((ht, hq))
                sh = []
                for b in range(KS):
                    ht, hq = hs[b]
                    ht.wait()
                    hq.wait()

                    def mrow(r, c2, b=b):
                        for qq in range(DN // L):
                            sl2 = pl.ds(qq * L, L)
                            tb[b][r, sl2] = jnp.maximum(
                                tb[b][r, sl2] + qb[b][r, sl2], 0.0)
                        return c2

                    lax.fori_loop(0, CS, mrow, 0)
                    sh.append(pltpu.async_copy(
                        tb[b], acc.at[iv.at[c0 + b]], ss[b], add=True))
                for x in sh:
                    x.wait()
                return carry

            lax.fori_loop(0, SLABS // KS, group, 0)
        plsc.subcore_barrier()

        @pl.when(sid < NS - 1)
        def _exp_main():
            def eblk(b, carry):
                row0 = sid * 624 + b * 104
                pltpu.sync_copy(acc.at[pl.ds(row0, 104)],
                                out_h.at[pl.ds(cid * N + row0, 104)])
                return carry
            lax.fori_loop(0, 6, eblk, 0)

        @pl.when(sid == NS - 1)
        def _exp_tail():
            def eblk(b, carry):
                row0 = 9360 + b * 80
                pltpu.sync_copy(acc.at[pl.ds(row0, 80)],
                                out_h.at[pl.ds(cid * N + row0, 80)])
                return carry
            lax.fori_loop(0, 8, eblk, 0)

    return k(q, t, *idx_slabs)


_NBLK = 5000
_EBLK = 8000


def _tc_node(x, ne_w1, ne_b1, ne_w2, ne_b2, wab, wq):
    def body(x_r, w1_r, b1_r, w2_r, b2_r, wab_r, wq_r, pab_r, q_r):
        h = jnp.maximum(jnp.dot(x_r[...], w1_r[...],
                                preferred_element_type=jnp.float32) + b1_r[...], 0.0)
        nf = jnp.dot(h, w2_r[...], preferred_element_type=jnp.float32) + b2_r[...]
        pab_r[...] = jnp.dot(nf, wab_r[...], preferred_element_type=jnp.float32)
        q_r[...] = jnp.dot(nf, wq_r[...], preferred_element_type=jnp.float32)

    full = lambda s: pl.BlockSpec(s, lambda i: (0, 0))
    return pl.pallas_call(
        body,
        grid=(N // _NBLK,),
        in_specs=[
            pl.BlockSpec((_NBLK, DN), lambda i: (i, 0)),
            full((DN, 128)), full((1, 128)), full((128, DN)), full((1, DN)),
            full((DN, DN)), full((DN, DN)),
        ],
        out_specs=[
            pl.BlockSpec((_NBLK, DN), lambda i: (i, 0)),
            pl.BlockSpec((_NBLK, DN), lambda i: (i, 0)),
        ],
        out_shape=[
            jax.ShapeDtypeStruct((N, DN), jnp.float32),
            jax.ShapeDtypeStruct((N, DN), jnp.float32),
        ],
    )(x, ne_w1, ne_b1, ne_w2, ne_b2, wab, wq)


def _tc_edge_first(s0, edge_attr, ee_w1, ee_b1, ee_w2, ee_b2,
                   wc, me_b1, me_w2, me_b2, wt, mn_b1):
    def body(s_r, ea_r, w1_r, b1_r, w2_r, b2_r, wc_r, mb1_r, mw2_r, mb2_r,
             wt_r, nb1_r, ef_r, t_r):
        e1 = jnp.maximum(jnp.dot(ea_r[...], w1_r[...],
                                 preferred_element_type=jnp.float32) + b1_r[...], 0.0)
        ef0 = jnp.dot(e1, w2_r[...], preferred_element_type=jnp.float32) + b2_r[...]
        h1 = jnp.maximum(s_r[...].astype(jnp.float32)
                         + jnp.dot(ef0, wc_r[...],
                                   preferred_element_type=jnp.float32)
                         + mb1_r[...], 0.0)
        ef1 = jnp.maximum(jnp.dot(h1, mw2_r[...],
                                  preferred_element_type=jnp.float32) + mb2_r[...], 0.0)
        ef_r[...] = ef1
        t_r[...] = (jnp.dot(ef1, wt_r[...], preferred_element_type=jnp.float32)
                    + nb1_r[...])

    full = lambda s: pl.BlockSpec(s, lambda i: (0, 0))
    return pl.pallas_call(
        body,
        grid=(E // _EBLK,),
        in_specs=[
            pl.BlockSpec((_EBLK, 64), lambda i: (i, 0)),
            pl.BlockSpec((_EBLK, DE), lambda i: (i, 0)),
            full((DE, 64)), full((1, 64)), full((64, DE)), full((1, DE)),
            full((DE, 64)), full((1, 64)), full((64, DE)), full((1, DE)),
            full((DE, DN)), full((1, DN)),
        ],
        out_specs=[
            pl.BlockSpec((_EBLK, DE), lambda i: (i, 0)),
            pl.BlockSpec((_EBLK, DN), lambda i: (i, 0)),
        ],
        out_shape=[
            jax.ShapeDtypeStruct((E, DE), jnp.float32),
            jax.ShapeDtypeStruct((E, DN), jnp.float32),
        ],
    )(s0, edge_attr, ee_w1, ee_b1, ee_w2, ee_b2, wc, me_b1, me_w2, me_b2,
      wt, mn_b1)


def _tc_combine(parts, wab):
    def body(p0_r, p1_r, wab_r, pab_r):
        nf = p0_r[...] + p1_r[...]
        pab_r[...] = jnp.dot(nf, wab_r[...], preferred_element_type=jnp.float32)

    return pl.pallas_call(
        body,
        grid=(N // _NBLK,),
        in_specs=[
            pl.BlockSpec((_NBLK, DN), lambda i: (i, 0)),
            pl.BlockSpec((_NBLK, DN), lambda i: (i + N // _NBLK, 0)),
            pl.BlockSpec((DN, DN), lambda i: (0, 0)),
        ],
        out_specs=pl.BlockSpec((_NBLK, DN), lambda i: (i, 0)),
        out_shape=jax.ShapeDtypeStruct((N, DN), jnp.float32),
    )(parts, parts, wab)


def _tc_edge_last(s1, ef1, wc, me_b1, me_w2, me_b2, cl_w1, cl_b1, cl_w2, cl_b2):
    def body(s_r, ef_r, wc_r, mb1_r, mw2_r, mb2_r, cw1_r, cb1_r, cw2_r, cb2_r,
             o_r):
        h1 = jnp.maximum(s_r[...].astype(jnp.float32)
                         + jnp.dot(ef_r[...], wc_r[...],
                                   preferred_element_type=jnp.float32)
                         + mb1_r[...], 0.0)
        ef2 = jnp.maximum(jnp.dot(h1, mw2_r[...],
                                  preferred_element_type=jnp.float32) + mb2_r[...], 0.0)
        c1 = jnp.maximum(jnp.dot(ef2, cw1_r[...],
                                 preferred_element_type=jnp.float32) + cb1_r[...], 0.0)
        o_r[...] = jnp.dot(c1, cw2_r[...],
                           preferred_element_type=jnp.float32) + cb2_r[...]

    full = lambda s: pl.BlockSpec(s, lambda i: (0, 0))
    return pl.pallas_call(
        body,
        grid=(E // _EBLK,),
        in_specs=[
            pl.BlockSpec((_EBLK, 64), lambda i: (i, 0)),
            pl.BlockSpec((_EBLK, DE), lambda i: (i, 0)),
            full((DE, 64)), full((1, 64)), full((64, DE)), full((1, DE)),
            full((DE, 32)), full((1, 32)), full((32, 1)), full((1, 1)),
        ],
        out_specs=pl.BlockSpec((_EBLK, 1), lambda i: (i, 0)),
        out_shape=jax.ShapeDtypeStruct((E, 1), jnp.float32),
    )(s1, ef1, wc, me_b1, me_w2, me_b2, cl_w1, cl_b1, cl_w2, cl_b2)


def kernel(x, edge_attr, edge_index, ne_w1, ne_b1, ne_w2, ne_b2,
           ee_w1, ee_b1, ee_w2, ee_b2, me_w1, me_b1, me_w2, me_b2,
           mn_w1, mn_b1, cl_w1, cl_b1, cl_w2, cl_b2):
    idx_j4 = edge_index[0].reshape(NW, NSLAB, SLAB, C)
    idx_i4 = edge_index[1].reshape(NW, NSLAB, SLAB, C)
    jj_slabs = tuple(idx_j4[:, s] for s in range(NSLAB))
    ii_slabs = tuple(idx_i4[:, s] for s in range(NSLAB))
    idx_i4s = edge_index[1].reshape(NW, NSLAB, SLABS, CS)
    ii_slabs_s = tuple(idx_i4s[:, s] for s in range(NSLAB))

    wab = jnp.concatenate([me_w1[:DN], me_w1[DN:2 * DN]], axis=1)
    wc = me_w1[2 * DN:]
    wq = mn_w1[:DN]
    wt = mn_w1[DN:]
    r2 = lambda b: b.reshape(1, -1)

    pab0, q0 = _tc_node(x, ne_w1, r2(ne_b1), ne_w2, r2(ne_b2), wab, wq)
    s0 = _sc_gather_sum(pab0, ii_slabs, jj_slabs)
    ef1, t0 = _tc_edge_first(s0, edge_attr, ee_w1, r2(ee_b1), ee_w2, r2(ee_b2),
                             wc, r2(me_b1), me_w2, r2(me_b2), wt, r2(mn_b1))
    parts = _sc_scatter_msg(q0, t0, ii_slabs_s)
    pab1 = _tc_combine(parts, wab)
    s1 = _sc_gather_sum(pab1, ii_slabs, jj_slabs)
    out = _tc_edge_last(s1, ef1, wc, r2(me_b1), me_w2, r2(me_b2),
                        cl_w1, r2(cl_b1), cl_w2, r2(cl_b2))
    return out


# speedup vs baseline: 1.1008x; 1.0150x over previous
import functools

import jax
import jax.numpy as jnp
from jax import lax
from jax.experimental import pallas as pl
from jax.experimental.pallas import tpu as pltpu
from jax.experimental.pallas import tpu_sc as plsc

N = 10000
E = 320000
DN = 128
DE = 16

NC = 2
NS = 16
L = 16
NW = NC * NS
EPW = E // NW
C = 40
NCH = EPW // C
KG = 5
NSLAB = 5
SLAB = NCH // NSLAB
CS = 16
NCHS = EPW // CS
KS = 5
SLABS = NCHS // NSLAB

_mesh = lambda: plsc.VectorSubcoreMesh(core_axis_name="c", subcore_axis_name="s")


def _sc_gather_sum(pab, ii_slabs, jj_slabs):
    @functools.partial(
        pl.kernel,
        mesh=_mesh(),
        out_type=jax.ShapeDtypeStruct((E, 64), jnp.float32),
        scratch_types=(
            [pltpu.VMEM((SLAB, C), jnp.int32)] * 2
            + [pltpu.VMEM((C, DN), jnp.float32)] * (2 * KG)
            + [pltpu.VMEM((C, 64), jnp.float32)] * KG
            + [pltpu.SemaphoreType.DMA] * (3 * KG)
        ),
    )
    def k(pab_h, *args):
        ii_h = args[0:NSLAB]
        jj_h = args[NSLAB:2 * NSLAB]
        s_h = args[2 * NSLAB]
        iv, jv = args[2 * NSLAB + 1:2 * NSLAB + 3]
        bufs = args[2 * NSLAB + 3:]
        ab = bufs[0:KG]
        bb = bufs[KG:2 * KG]
        sb = bufs[2 * KG:3 * KG]
        sga = bufs[3 * KG:4 * KG]
        sgb = bufs[4 * KG:5 * KG]
        sst = bufs[5 * KG:6 * KG]
        wid = lax.axis_index("c") * NS + lax.axis_index("s")
        ebase = wid * EPW

        for sl in range(NSLAB):
            pltpu.sync_copy(ii_h[sl].at[wid], iv)
            pltpu.sync_copy(jj_h[sl].at[wid], jv)
            cbase = ebase + sl * SLAB * C

            def group(g, carry, cbase=cbase):
                c0 = g * KG
                hs = []
                for b in range(KG):
                    ha = pltpu.async_copy(pab_h.at[iv.at[c0 + b]], ab[b], sga[b])
                    hb = pltpu.async_copy(pab_h.at[jv.at[c0 + b]], bb[b], sgb[b])
                    hs.append((ha, hb))
                sh = []
                for b in range(KG):
                    ha, hb = hs[b]
                    ha.wait()
                    hb.wait()

                    def add_row(r, c2, b=b):
                        for g2 in range(64 // L):
                            sl2 = pl.ds(g2 * L, L)
                            sb[b][r, sl2] = (ab[b][r, sl2]
                                             + bb[b][r, pl.ds(64 + g2 * L, L)])
                        return c2

                    lax.fori_loop(0, C, add_row, 0)
                    sh.append(pltpu.async_copy(
                        sb[b], s_h.at[pl.ds(cbase + (c0 + b) * C, C)], sst[b]))
                for x in sh:
                    x.wait()
                return carry

            lax.fori_loop(0, SLAB // KG, group, 0)

    return k(pab, *ii_slabs, *jj_slabs)


def _sc_scatter_msg(q, t, idx_slabs):
    @functools.partial(
        pl.kernel,
        mesh=_mesh(),
        out_type=jax.ShapeDtypeStruct((2 * N, DN), jnp.float32),
        scratch_types=(
            [pltpu.VMEM((SLABS, CS), jnp.int32)]
            + [pltpu.VMEM((CS, DN), jnp.float32)] * (2 * KS)
            + [pltpu.VMEM((16, DN), jnp.float32)]
            + [pltpu.VMEM_SHARED((N, DN), jnp.float32)]
            + [pltpu.SemaphoreType.DMA] * (3 * KS)
        ),
    )
    def k(q_h, t_h, ii0, ii1, ii2, ii3_, ii4, out_h, iv, *bufs):
        ii_slabs = (ii0, ii1, ii2, ii3_, ii4)
        tb = bufs[0:KS]
        qb = bufs[KS:2 * KS]
        zbuf = bufs[2 * KS]
        acc = bufs[2 * KS + 1]
        st = bufs[2 * KS + 2:3 * KS + 2]
        sq = bufs[3 * KS + 2:4 * KS + 2]
        ss = bufs[4 * KS + 2:5 * KS + 2]
        cid = lax.axis_index("c")
        sid = lax.axis_index("s")
        wid = cid * NS + sid
        ebase = wid * EPW

        def zrow(r, c2):
            def zcol(qq, c3):
                zbuf[r, pl.ds(qq * L, L)] = jnp.zeros((L,), jnp.float32)
                return c3
            return lax.fori_loop(0, DN // L, zcol, c2)

        lax.fori_loop(0, 16, zrow, 0)

        @pl.when(sid < NS - 1)
        def _zero_main():
            def zblk(b, carry):
                pltpu.sync_copy(zbuf, acc.at[pl.ds(sid * 624 + b * 16, 16)])
                return carry
            lax.fori_loop(0, 39, zblk, 0)

        @pl.when(sid == NS - 1)
        def _zero_tail():
            def zblk(b, carry):
                pltpu.sync_copy(zbuf, acc.at[pl.ds(9360 + b * 16, 16)])
                return carry
            lax.fori_loop(0, 40, zblk, 0)

        plsc.subcore_barrier()

        for sl in range(NSLAB):
            pltpu.sync_copy(ii_slabs[sl].at[wid], iv)
            cbase = ebase + sl * SLABS * CS

            def group(g, carry, cbase=cbase):
                c0 = g * KS
                hs = []
                for b in range(KS):
                    ht = pltpu.async_copy(
                        t_h.at[pl.ds(cbase + (c0 + b) * CS, CS)], tb[b], st[b])
                    hq = pltpu.async_copy(q_h.at[iv.at[c0 + b]], qb[b], sq[b])
                    hs.append((ht, hq))
                sh = []
                for b in range(KS):
                    ht, hq = hs[b]
                    ht.wait()
                    hq.wait()

                    def mrow(r, c2, b=b):
                        for qq in range(DN // L):
                            sl2 = pl.ds(qq * L, L)
                            tb[b][r, sl2] = jnp.maximum(
                                tb[b][r, sl2] + qb[b][r, sl2], 0.0)
                        return c2

                    lax.fori_loop(0, CS, mrow, 0)
                    sh.append(pltpu.async_copy(
                        tb[b], acc.at[iv.at[c0 + b]], ss[b], add=True))
                for x in sh:
                    x.wait()
                return carry

            lax.fori_loop(0, SLABS // KS, group, 0)
        plsc.subcore_barrier()

        @pl.when(sid < NS - 1)
        def _exp_main():
            def eblk(b, carry):
                row0 = sid * 624 + b * 104
                pltpu.sync_copy(acc.at[pl.ds(row0, 104)],
                                out_h.at[pl.ds(cid * N + row0, 104)])
                return carry
            lax.fori_loop(0, 6, eblk, 0)

        @pl.when(sid == NS - 1)
        def _exp_tail():
            def eblk(b, carry):
                row0 = 9360 + b * 80
                pltpu.sync_copy(acc.at[pl.ds(row0, 80)],
                                out_h.at[pl.ds(cid * N + row0, 80)])
                return carry
            lax.fori_loop(0, 8, eblk, 0)

    return k(q, t, *idx_slabs)


_NBLK = 5000
_EBLK = 8000


def _tc_node(x, ne_w1, ne_b1, ne_w2, ne_b2, wab, wq):
    def body(x_r, w1_r, b1_r, w2_r, b2_r, wab_r, wq_r, pab_r, q_r):
        h = jnp.maximum(jnp.dot(x_r[...], w1_r[...],
                                preferred_element_type=jnp.float32) + b1_r[...], 0.0)
        nf = jnp.dot(h, w2_r[...], preferred_element_type=jnp.float32) + b2_r[...]
        pab_r[...] = jnp.dot(nf, wab_r[...], preferred_element_type=jnp.float32)
        q_r[...] = jnp.dot(nf, wq_r[...], preferred_element_type=jnp.float32)

    full = lambda s: pl.BlockSpec(s, lambda i: (0, 0))
    return pl.pallas_call(
        body,
        grid=(N // _NBLK,),
        in_specs=[
            pl.BlockSpec((_NBLK, DN), lambda i: (i, 0)),
            full((DN, 128)), full((1, 128)), full((128, DN)), full((1, DN)),
            full((DN, DN)), full((DN, DN)),
        ],
        out_specs=[
            pl.BlockSpec((_NBLK, DN), lambda i: (i, 0)),
            pl.BlockSpec((_NBLK, DN), lambda i: (i, 0)),
        ],
        out_shape=[
            jax.ShapeDtypeStruct((N, DN), jnp.float32),
            jax.ShapeDtypeStruct((N, DN), jnp.float32),
        ],
    )(x, ne_w1, ne_b1, ne_w2, ne_b2, wab, wq)


def _tc_edge_first(s0, edge_attr, ee_w1, ee_b1, ee_w2, ee_b2,
                   wc, me_b1, me_w2, me_b2, wt, mn_b1):
    def body(s_r, ea_r, w1_r, b1_r, w2_r, b2_r, wc_r, mb1_r, mw2_r, mb2_r,
             wt_r, nb1_r, ef_r, t_r):
        e1 = jnp.maximum(jnp.dot(ea_r[...], w1_r[...],
                                 preferred_element_type=jnp.float32) + b1_r[...], 0.0)
        ef0 = jnp.dot(e1, w2_r[...], preferred_element_type=jnp.float32) + b2_r[...]
        h1 = jnp.maximum(s_r[...].astype(jnp.float32)
                         + jnp.dot(ef0, wc_r[...],
                                   preferred_element_type=jnp.float32)
                         + mb1_r[...], 0.0)
        ef1 = jnp.maximum(jnp.dot(h1, mw2_r[...],
                                  preferred_element_type=jnp.float32) + mb2_r[...], 0.0)
        ef_r[...] = ef1
        t_r[...] = (jnp.dot(ef1, wt_r[...], preferred_element_type=jnp.float32)
                    + nb1_r[...])

    full = lambda s: pl.BlockSpec(s, lambda i: (0, 0))
    return pl.pallas_call(
        body,
        grid=(E // _EBLK,),
        in_specs=[
            pl.BlockSpec((_EBLK, 64), lambda i: (i, 0)),
            pl.BlockSpec((_EBLK, DE), lambda i: (i, 0)),
            full((DE, 64)), full((1, 64)), full((64, DE)), full((1, DE)),
            full((DE, 64)), full((1, 64)), full((64, DE)), full((1, DE)),
            full((DE, DN)), full((1, DN)),
        ],
        out_specs=[
            pl.BlockSpec((_EBLK, DE), lambda i: (i, 0)),
            pl.BlockSpec((_EBLK, DN), lambda i: (i, 0)),
        ],
        out_shape=[
            jax.ShapeDtypeStruct((E, DE), jnp.float32),
            jax.ShapeDtypeStruct((E, DN), jnp.float32),
        ],
    )(s0, edge_attr, ee_w1, ee_b1, ee_w2, ee_b2, wc, me_b1, me_w2, me_b2,
      wt, mn_b1)


def _tc_combine(parts, wab):
    def body(p0_r, p1_r, wab_r, pab_r):
        nf = p0_r[...] + p1_r[...]
        pab_r[...] = jnp.dot(nf, wab_r[...], preferred_element_type=jnp.float32)

    return pl.pallas_call(
        body,
        grid=(N // _NBLK,),
        in_specs=[
            pl.BlockSpec((_NBLK, DN), lambda i: (i, 0)),
            pl.BlockSpec((_NBLK, DN), lambda i: (i + N // _NBLK, 0)),
            pl.BlockSpec((DN, DN), lambda i: (0, 0)),
        ],
        out_specs=pl.BlockSpec((_NBLK, DN), lambda i: (i, 0)),
        out_shape=jax.ShapeDtypeStruct((N, DN), jnp.float32),
    )(parts, parts, wab)


def _tc_edge_last(s1, ef1, wc, me_b1, me_w2, me_b2, cl_w1, cl_b1, cl_w2, cl_b2):
    def body(s_r, ef_r, wc_r, mb1_r, mw2_r, mb2_r, cw1_r, cb1_r, cw2_r, cb2_r,
             o_r):
        h1 = jnp.maximum(s_r[...].astype(jnp.float32)
                         + jnp.dot(ef_r[...], wc_r[...],
                                   preferred_element_type=jnp.float32)
                         + mb1_r[...], 0.0)
        ef2 = jnp.maximum(jnp.dot(h1, mw2_r[...],
                                  preferred_element_type=jnp.float32) + mb2_r[...], 0.0)
        c1 = jnp.maximum(jnp.dot(ef2, cw1_r[...],
                                 preferred_element_type=jnp.float32) + cb1_r[...], 0.0)
        o_r[...] = jnp.dot(c1, cw2_r[...],
                           preferred_element_type=jnp.float32) + cb2_r[...]

    full = lambda s: pl.BlockSpec(s, lambda i: (0, 0))
    return pl.pallas_call(
        body,
        grid=(E // _EBLK,),
        in_specs=[
            pl.BlockSpec((_EBLK, 64), lambda i: (i, 0)),
            pl.BlockSpec((_EBLK, DE), lambda i: (i, 0)),
            full((DE, 64)), full((1, 64)), full((64, DE)), full((1, DE)),
            full((DE, 32)), full((1, 32)), full((32, 1)), full((1, 1)),
        ],
        out_specs=pl.BlockSpec((_EBLK, 1), lambda i: (i, 0)),
        out_shape=jax.ShapeDtypeStruct((E, 1), jnp.float32),
    )(s1, ef1, wc, me_b1, me_w2, me_b2, cl_w1, cl_b1, cl_w2, cl_b2)


def kernel(x, edge_attr, edge_index, ne_w1, ne_b1, ne_w2, ne_b2,
           ee_w1, ee_b1, ee_w2, ee_b2, me_w1, me_b1, me_w2, me_b2,
           mn_w1, mn_b1, cl_w1, cl_b1, cl_w2, cl_b2):
    idx_j4 = edge_index[0].reshape(NW, NSLAB, SLAB, C)
    idx_i4 = edge_index[1].reshape(NW, NSLAB, SLAB, C)
    jj_slabs = tuple(idx_j4[:, s] for s in range(NSLAB))
    ii_slabs = tuple(idx_i4[:, s] for s in range(NSLAB))
    idx_i4s = edge_index[1].reshape(NW, NSLAB, SLABS, CS)
    ii_slabs_s = tuple(idx_i4s[:, s] for s in range(NSLAB))

    wab = jnp.concatenate([me_w1[:DN], me_w1[DN:2 * DN]], axis=1)
    wc = me_w1[2 * DN:]
    wq = mn_w1[:DN]
    wt = mn_w1[DN:]
    r2 = lambda b: b.reshape(1, -1)

    pab0, q0 = _tc_node(x, ne_w1, r2(ne_b1), ne_w2, r2(ne_b2), wab, wq)
    s0 = _sc_gather_sum(pab0, ii_slabs, jj_slabs)
    ef1, t0 = _tc_edge_first(s0, edge_attr, ee_w1, r2(ee_b1), ee_w2, r2(ee_b2),
                             wc, r2(me_b1), me_w2, r2(me_b2), wt, r2(mn_b1))
    parts = _sc_scatter_msg(q0, t0, ii_slabs_s)
    pab1 = _tc_combine(parts, wab)
    s1 = _sc_gather_sum(pab1, ii_slabs, jj_slabs)
    out = _tc_edge_last(s1, ef1, wc, r2(me_b1), me_w2, r2(me_b2),
                        cl_w1, r2(cl_b1), cl_w2, r2(cl_b2))
    return out
